# baseline (speedup 1.0000x reference)
import functools

import jax
import jax.numpy as jnp
from jax import lax
from jax.experimental import pallas as pl
from jax.experimental.pallas import tpu as pltpu
from jax.experimental.pallas import tpu_sc as plsc

B = 1024
L = 200
EMBED_DIM = 64
NUM_ENT = 1000000
NUM_PAIRS_TBL = NUM_ENT // 2

NUM_CORES = 2
NUM_SUBCORES = 16
NUM_WORKERS = NUM_CORES * NUM_SUBCORES
LANES = 16

BBLK = 256
NUM_BLOCKS = L * 2 * (B // BBLK)
BPW = NUM_BLOCKS // NUM_WORKERS
NCH = BBLK // LANES


def _gather_native(trip_t, r2):
    mesh = plsc.VectorSubcoreMesh(core_axis_name="c", subcore_axis_name="s")

    @functools.partial(
        pl.kernel,
        mesh=mesh,
        out_type=jax.ShapeDtypeStruct((L, 2, EMBED_DIM, B), jnp.float32),
        compiler_params=pltpu.CompilerParams(
            needs_layout_passes=False, use_tc_tiling_on_sc=True
        ),
        scratch_types=[
            pltpu.VMEM((2, 2, 128), jnp.int32),
            pltpu.VMEM((2, 2, 128), jnp.int32),
            pltpu.VMEM((2, BBLK), jnp.int32),
            pltpu.VMEM((2, BBLK, 128), jnp.float32),
            pltpu.VMEM((2, EMBED_DIM, BBLK), jnp.float32),
            pltpu.SemaphoreType.DMA,
            pltpu.SemaphoreType.DMA,
            pltpu.SemaphoreType.DMA,
            pltpu.SemaphoreType.DMA,
            pltpu.SemaphoreType.DMA,
        ],
    )
    def k(trip_hbm, r2_hbm, o2_hbm, idxr_v, idxp_v, cb_v, g_v, o_v,
          isem0, isem1, gsem, osem0, osem1):
        wid = lax.axis_index("s") * NUM_CORES + lax.axis_index("c")
        lane = lax.iota(jnp.int32, LANES)
        isems = (isem0, isem1)
        osems = (osem0, osem1)

        def parts(t):
            bid = wid * BPW + t
            return bid // 8, (bid // 4) & 1, bid & 3

        def fire_idx(t, b):
            l, j, bb = parts(t)
            for h in range(2):
                pltpu.async_copy(
                    trip_hbm.at[2 * j, l, pl.ds(BBLK * bb + 128 * h, 128)],
                    idxr_v.at[b, h], isems[b],
                )

        def wait_idx(b):
            for h in range(2):
                pltpu.make_async_copy(
                    trip_hbm.at[0, 0, pl.ds(0, 128)], idxr_v.at[b, h], isems[b]
                ).wait()

        def prep_fire_gather(b):
            for h in range(2):
                for c in range(8):
                    v = idxr_v[b, h, pl.ds(c * LANES, LANES)]
                    idxp_v[b, h, pl.ds(c * LANES, LANES)] = v >> 1
                    cb_v[b, pl.ds(h * 128 + c * LANES, LANES)] = (v & 1) << 6
            for h in range(2):
                pltpu.async_copy(
                    r2_hbm.at[idxp_v.at[b, h]],
                    g_v.at[b, pl.ds(h * 128, 128), :], gsem,
                )

        def wait_gather(b):
            for h in range(2):
                pltpu.make_async_copy(
                    r2_hbm.at[pl.ds(0, 128), :],
                    g_v.at[b, pl.ds(h * 128, 128), :], gsem,
                ).wait()

        def transpose(b):
            def chunk(c, _):
                e16 = lane + c * LANES
                cb16 = cb_v[b, pl.ds(c * LANES, LANES)]
                col0 = cb16
                for d in range(EMBED_DIM):
                    o_v[b, d, pl.ds(c * LANES, LANES)] = plsc.load_gather(
                        g_v, [jnp.full((LANES,), b, jnp.int32), e16, col0 + d]
                    )
                return _
            lax.fori_loop(0, NCH, chunk, None)

        def fire_store(t, b):
            l, j, bb = parts(t)
            pltpu.async_copy(
                o_v.at[b], o2_hbm.at[l, j, :, pl.ds(BBLK * bb, BBLK)],
                osems[b],
            )

        def wait_store(t, b):
            l, j, bb = parts(t)
            pltpu.make_async_copy(
                o_v.at[b], o2_hbm.at[l, j, :, pl.ds(BBLK * bb, BBLK)],
                osems[b],
            ).wait()

        fire_idx(0, 0)
        fire_idx(1, 1)
        wait_idx(0)
        prep_fire_gather(0)

        def body(u, _):
            for b in range(2):
                t = 2 * u + b
                nb = 1 - b
                wait_gather(b)
                pl.when(t + 2 < BPW)(lambda t=t, b=b: fire_idx(t + 2, b))

                def mid(t=t, nb=nb):
                    wait_idx(nb)
                    prep_fire_gather(nb)
                pl.when(t + 1 < BPW)(mid)
                pl.when(t >= 2)(lambda t=t, b=b: wait_store(t - 2, b))
                transpose(b)
                fire_store(t, b)
            return _

        lax.fori_loop(0, BPW // 2, body, None)
        wait_store(BPW - 2, 0)
        wait_store(BPW - 1, 1)

    return k(trip_t, r2)


def kernel(triples, emb_table):
    r2 = jnp.concatenate(
        [emb_table[0:NUM_ENT:2], emb_table[1:NUM_ENT:2]], axis=1
    )
    trip_t = jnp.transpose(triples, (2, 1, 0))
    o2 = _gather_native(trip_t, r2)
    return jnp.transpose(o2, (3, 0, 1, 2))


# speedup vs baseline: 7.6227x; 7.6227x over previous
import functools

import jax
import jax.numpy as jnp
from jax import lax
from jax.experimental import pallas as pl
from jax.experimental.pallas import tpu as pltpu
from jax.experimental.pallas import tpu_sc as plsc

B = 1024
L = 200
EMBED_DIM = 64
NUM_ENT = 1000000
NUM_PAIRS_TBL = NUM_ENT // 2

NUM_CORES = 2
NUM_SUBCORES = 16
NUM_WORKERS = NUM_CORES * NUM_SUBCORES
LANES = 16

BBLK = 256
NUM_BLOCKS = L * 2 * (B // BBLK)
BPW = NUM_BLOCKS // NUM_WORKERS
NCH = BBLK // LANES


def _gather_native(trip_t, r2):
    mesh = plsc.VectorSubcoreMesh(core_axis_name="c", subcore_axis_name="s")

    @functools.partial(
        pl.kernel,
        mesh=mesh,
        out_type=jax.ShapeDtypeStruct((L, 2, EMBED_DIM, B), jnp.float32),
        compiler_params=pltpu.CompilerParams(
            needs_layout_passes=False, use_tc_tiling_on_sc=True
        ),
        scratch_types=[
            pltpu.VMEM((2, 2, 128), jnp.int32),
            pltpu.VMEM((2, 2, 128), jnp.int32),
            pltpu.VMEM((2, BBLK), jnp.int32),
            pltpu.VMEM((2, BBLK, 128), jnp.float32),
            pltpu.VMEM((2, EMBED_DIM, BBLK), jnp.float32),
            pltpu.SemaphoreType.DMA,
            pltpu.SemaphoreType.DMA,
            pltpu.SemaphoreType.DMA,
            pltpu.SemaphoreType.DMA,
            pltpu.SemaphoreType.DMA,
        ],
    )
    def k(trip_hbm, r2_hbm, o2_hbm, idxr_v, idxp_v, cb_v, g_v, o_v,
          isem0, isem1, gsem, osem0, osem1):
        wid = lax.axis_index("s") * NUM_CORES + lax.axis_index("c")
        lane = lax.iota(jnp.int32, LANES)
        isems = (isem0, isem1)
        osems = (osem0, osem1)

        def parts(t):
            bid = wid * BPW + t
            return bid // 8, (bid // 4) & 1, bid & 3

        def fire_idx(t, b):
            l, j, bb = parts(t)
            for h in range(2):
                pltpu.async_copy(
                    trip_hbm.at[2 * j, l, pl.ds(BBLK * bb + 128 * h, 128)],
                    idxr_v.at[b, h], isems[b],
                )

        def wait_idx(b):
            for h in range(2):
                pltpu.make_async_copy(
                    trip_hbm.at[0, 0, pl.ds(0, 128)], idxr_v.at[b, h], isems[b]
                ).wait()

        def prep_fire_gather(b):
            for h in range(2):
                for c in range(8):
                    v = idxr_v[b, h, pl.ds(c * LANES, LANES)]
                    idxp_v[b, h, pl.ds(c * LANES, LANES)] = v >> 1
                    cb_v[b, pl.ds(h * 128 + c * LANES, LANES)] = (v & 1) << 6
            for h in range(2):
                pltpu.async_copy(
                    r2_hbm.at[idxp_v.at[b, h]],
                    g_v.at[b, pl.ds(h * 128, 128), :], gsem,
                )

        def wait_gather(b):
            for h in range(2):
                pltpu.make_async_copy(
                    r2_hbm.at[pl.ds(0, 128), :],
                    g_v.at[b, pl.ds(h * 128, 128), :], gsem,
                ).wait()

        def transpose(b):
            def chunk(c, _):
                e16 = lane + c * LANES
                cb16 = cb_v[b, pl.ds(c * LANES, LANES)]
                col0 = cb16
                for d in range(EMBED_DIM):
                    o_v[b, d, pl.ds(c * LANES, LANES)] = plsc.load_gather(
                        g_v, [jnp.full((LANES,), b, jnp.int32), e16, col0 + d]
                    )
                return _
            lax.fori_loop(0, NCH, chunk, None)

        def fire_store(t, b):
            l, j, bb = parts(t)
            pltpu.async_copy(
                o_v.at[b], o2_hbm.at[l, j, :, pl.ds(BBLK * bb, BBLK)],
                osems[b],
            )

        def wait_store(t, b):
            l, j, bb = parts(t)
            pltpu.make_async_copy(
                o_v.at[b], o2_hbm.at[l, j, :, pl.ds(BBLK * bb, BBLK)],
                osems[b],
            ).wait()

        fire_idx(0, 0)
        fire_idx(1, 1)
        wait_idx(0)
        prep_fire_gather(0)

        def body(u, _):
            for b in range(2):
                t = 2 * u + b
                nb = 1 - b
                wait_gather(b)
                pl.when(t + 2 < BPW)(lambda t=t, b=b: fire_idx(t + 2, b))

                def mid(t=t, nb=nb):
                    wait_idx(nb)
                    prep_fire_gather(nb)
                pl.when(t + 1 < BPW)(mid)
                pl.when(t >= 2)(lambda t=t, b=b: wait_store(t - 2, b))
                transpose(b)
                fire_store(t, b)
            return _

        lax.fori_loop(0, BPW // 2, body, None)
        wait_store(BPW - 2, 0)
        wait_store(BPW - 1, 1)

    return k(trip_t, r2)


def kernel(triples, emb_table):
    r2 = emb_table[:NUM_ENT].reshape(NUM_PAIRS_TBL, 128)
    trip_t = jnp.transpose(triples, (2, 1, 0))
    o2 = _gather_native(trip_t, r2)
    return jnp.transpose(o2, (3, 0, 1, 2))


# speedup vs baseline: 7.6296x; 1.0009x over previous
import functools

import jax
import jax.numpy as jnp
from jax import lax
from jax.experimental import pallas as pl
from jax.experimental.pallas import tpu as pltpu
from jax.experimental.pallas import tpu_sc as plsc

B = 1024
L = 200
EMBED_DIM = 64
NUM_ENT = 1000000
NUM_PAIRS_TBL = NUM_ENT // 2

NUM_CORES = 2
NUM_SUBCORES = 16
NUM_WORKERS = NUM_CORES * NUM_SUBCORES
LANES = 16

BBLK = 256
NUM_BLOCKS = L * 2 * (B // BBLK)
BPW = NUM_BLOCKS // NUM_WORKERS
NCH = BBLK // LANES


def _gather_native(trip_t, r2):
    mesh = plsc.VectorSubcoreMesh(core_axis_name="c", subcore_axis_name="s")

    @functools.partial(
        pl.kernel,
        mesh=mesh,
        out_type=jax.ShapeDtypeStruct((L, 2, EMBED_DIM, B), jnp.float32),
        compiler_params=pltpu.CompilerParams(
            needs_layout_passes=False, use_tc_tiling_on_sc=True
        ),
        scratch_types=[
            pltpu.VMEM((2, 2, 128), jnp.int32),
            pltpu.VMEM((2, 2, 128), jnp.int32),
            pltpu.VMEM((2, BBLK), jnp.int32),
            pltpu.VMEM((2, BBLK, 128), jnp.float32),
            pltpu.VMEM((2, EMBED_DIM, BBLK), jnp.float32),
            pltpu.SemaphoreType.DMA,
            pltpu.SemaphoreType.DMA,
            pltpu.SemaphoreType.DMA,
            pltpu.SemaphoreType.DMA,
            pltpu.SemaphoreType.DMA,
        ],
    )
    def k(trip_hbm, r2_hbm, o2_hbm, idxr_v, idxp_v, cb_v, g_v, o_v,
          isem0, isem1, gsem, osem0, osem1):
        wid = lax.axis_index("s") * NUM_CORES + lax.axis_index("c")
        lane = lax.iota(jnp.int32, LANES)
        isems = (isem0, isem1)
        osems = (osem0, osem1)

        def parts(t):
            bid = wid * BPW + t
            return bid // 8, (bid // 4) & 1, bid & 3

        def fire_idx(t, b):
            l, j, bb = parts(t)
            for h in range(2):
                pltpu.async_copy(
                    trip_hbm.at[2 * j, l, pl.ds(BBLK * bb + 128 * h, 128)],
                    idxr_v.at[b, h], isems[b],
                )

        def wait_idx(b):
            for h in range(2):
                pltpu.make_async_copy(
                    trip_hbm.at[0, 0, pl.ds(0, 128)], idxr_v.at[b, h], isems[b]
                ).wait()

        def prep_fire_gather(b):
            for h in range(2):
                for c in range(8):
                    v = idxr_v[b, h, pl.ds(c * LANES, LANES)]
                    idxp_v[b, h, pl.ds(c * LANES, LANES)] = v >> 1
                    cb_v[b, pl.ds(h * 128 + c * LANES, LANES)] = (v & 1) << 6
            for h in range(2):
                pltpu.async_copy(
                    r2_hbm.at[idxp_v.at[b, h]],
                    g_v.at[b, pl.ds(h * 128, 128), :], gsem,
                )

        def wait_gather(b):
            for h in range(2):
                pltpu.make_async_copy(
                    r2_hbm.at[pl.ds(0, 128), :],
                    g_v.at[b, pl.ds(h * 128, 128), :], gsem,
                ).wait()

        zero16 = lane * 0

        def transpose(b):
            def chunk(c, _):
                e16 = lane + c * LANES
                cb16 = cb_v[b, pl.ds(c * LANES, LANES)]
                fl = (e16 << 7) + (cb16 + b * BBLK * 128)
                for d in range(EMBED_DIM):
                    o_v[b, d, pl.ds(c * LANES, LANES)] = plsc.load_gather(
                        g_v, [zero16, zero16, fl + d]
                    )
                return _
            lax.fori_loop(0, NCH, chunk, None)

        def fire_store(t, b):
            l, j, bb = parts(t)
            pltpu.async_copy(
                o_v.at[b], o2_hbm.at[l, j, :, pl.ds(BBLK * bb, BBLK)],
                osems[b],
            )

        def wait_store(t, b):
            l, j, bb = parts(t)
            pltpu.make_async_copy(
                o_v.at[b], o2_hbm.at[l, j, :, pl.ds(BBLK * bb, BBLK)],
                osems[b],
            ).wait()

        fire_idx(0, 0)
        fire_idx(1, 1)
        wait_idx(0)
        prep_fire_gather(0)

        def body(u, _):
            for b in range(2):
                t = 2 * u + b
                nb = 1 - b
                wait_gather(b)
                pl.when(t + 2 < BPW)(lambda t=t, b=b: fire_idx(t + 2, b))

                def mid(t=t, nb=nb):
                    wait_idx(nb)
                    prep_fire_gather(nb)
                pl.when(t + 1 < BPW)(mid)
                pl.when(t >= 2)(lambda t=t, b=b: wait_store(t - 2, b))
                transpose(b)
                fire_store(t, b)
            return _

        lax.fori_loop(0, BPW // 2, body, None)
        wait_store(BPW - 2, 0)
        wait_store(BPW - 1, 1)

    return k(trip_t, r2)


def kernel(triples, emb_table):
    r2 = emb_table[:NUM_ENT].reshape(NUM_PAIRS_TBL, 128)
    trip_t = jnp.transpose(triples, (2, 1, 0))
    o2 = _gather_native(trip_t, r2)
    return jnp.transpose(o2, (3, 0, 1, 2))


# speedup vs baseline: 10.5653x; 1.3848x over previous
import functools

import jax
import jax.numpy as jnp
from jax import lax
from jax.experimental import pallas as pl
from jax.experimental.pallas import tpu as pltpu
from jax.experimental.pallas import tpu_sc as plsc

B = 1024
L = 200
EMBED_DIM = 64
NUM_ENT = 1000000
NUM_PAIRS_TBL = NUM_ENT // 2

NUM_CORES = 2
NUM_SUBCORES = 16
NUM_WORKERS = NUM_CORES * NUM_SUBCORES
LANES = 16

BBLK = 256
NUM_BLOCKS = L * 2 * (B // BBLK)
BPW = NUM_BLOCKS // NUM_WORKERS
NCH = BBLK // LANES


def _gather_native(trip_t, r2):
    mesh = plsc.VectorSubcoreMesh(core_axis_name="c", subcore_axis_name="s")

    @functools.partial(
        pl.kernel,
        mesh=mesh,
        out_type=jax.ShapeDtypeStruct((L, 2, EMBED_DIM, B), jnp.float32),
        compiler_params=pltpu.CompilerParams(
            needs_layout_passes=False, use_tc_tiling_on_sc=True
        ),
        scratch_types=[
            pltpu.VMEM((2, 2, 128), jnp.int32),
            pltpu.VMEM((2, 2, 128), jnp.int32),
            pltpu.VMEM((2, BBLK), jnp.int32),
            pltpu.VMEM((2, BBLK, 128), jnp.float32),
            pltpu.VMEM((2, 2 * EMBED_DIM, 128), jnp.float32),
            pltpu.SemaphoreType.DMA,
            pltpu.SemaphoreType.DMA,
            pltpu.SemaphoreType.DMA,
            pltpu.SemaphoreType.DMA,
            pltpu.SemaphoreType.DMA,
        ],
    )
    def k(trip_hbm, r2_hbm, o2_hbm, idxr_v, idxp_v, cb_v, g_v, o_v,
          isem0, isem1, gsem, osem0, osem1):
        wid = lax.axis_index("s") * NUM_CORES + lax.axis_index("c")
        lane = lax.iota(jnp.int32, LANES)
        isems = (isem0, isem1)
        osems = (osem0, osem1)

        def parts(t):
            bid = wid * BPW + t
            return bid // 8, (bid // 4) & 1, bid & 3

        def fire_idx(t, b):
            l, j, bb = parts(t)
            for h in range(2):
                pltpu.async_copy(
                    trip_hbm.at[2 * j, l, pl.ds(BBLK * bb + 128 * h, 128)],
                    idxr_v.at[b, h], isems[b],
                )

        def wait_idx(b):
            for h in range(2):
                pltpu.make_async_copy(
                    trip_hbm.at[0, 0, pl.ds(0, 128)], idxr_v.at[b, h], isems[b]
                ).wait()

        def prep_fire_gather(b):
            for h in range(2):
                for c in range(8):
                    v = idxr_v[b, h, pl.ds(c * LANES, LANES)]
                    idxp_v[b, h, pl.ds(c * LANES, LANES)] = v >> 1
                    cb_v[b, pl.ds(h * 128 + c * LANES, LANES)] = (v & 1) << 6
            for h in range(2):
                pltpu.async_copy(
                    r2_hbm.at[idxp_v.at[b, h]],
                    g_v.at[b, pl.ds(h * 128, 128), :], gsem,
                )

        def wait_gather(b):
            for h in range(2):
                pltpu.make_async_copy(
                    r2_hbm.at[pl.ds(0, 128), :],
                    g_v.at[b, pl.ds(h * 128, 128), :], gsem,
                ).wait()

        zero16 = lane * 0

        def transpose(b):
            def chunk(c, _):
                e16 = lane + c * LANES
                cb16 = cb_v[b, pl.ds(c * LANES, LANES)]
                el16 = e16 & 127
                eh64 = (c // 8) * 64
                gbase = (e16 << 7) + cb16 + b * (BBLK * 128)
                obase = el16 + (b * 128 + eh64) * 128
                for dc in range(EMBED_DIM // LANES):
                    d0 = dc * LANES
                    rbase = gbase + d0
                    wbase = obase + (d0 << 7)
                    rot = lane
                    for j in range(LANES):
                        val = plsc.load_gather(
                            g_v, [zero16, zero16, rbase + rot]
                        )
                        plsc.store_scatter(
                            o_v, [zero16, zero16, wbase + (rot << 7)], val
                        )
                        rot = (rot + 1) & 15
                return _
            lax.fori_loop(0, NCH, chunk, None)

        def fire_store(t, b):
            l, j, bb = parts(t)
            for eh in range(2):
                pltpu.async_copy(
                    o_v.at[b, pl.ds(eh * EMBED_DIM, EMBED_DIM)],
                    o2_hbm.at[l, j, :, pl.ds(BBLK * bb + 128 * eh, 128)],
                    osems[b],
                )

        def wait_store(t, b):
            l, j, bb = parts(t)
            for eh in range(2):
                pltpu.make_async_copy(
                    o_v.at[b, pl.ds(eh * EMBED_DIM, EMBED_DIM)],
                    o2_hbm.at[l, j, :, pl.ds(BBLK * bb + 128 * eh, 128)],
                    osems[b],
                ).wait()

        fire_idx(0, 0)
        fire_idx(1, 1)
        wait_idx(0)
        prep_fire_gather(0)

        def body(u, _):
            for b in range(2):
                t = 2 * u + b
                nb = 1 - b
                wait_gather(b)
                pl.when(t + 2 < BPW)(lambda t=t, b=b: fire_idx(t + 2, b))

                def mid(t=t, nb=nb):
                    wait_idx(nb)
                    prep_fire_gather(nb)
                pl.when(t + 1 < BPW)(mid)
                pl.when(t >= 2)(lambda t=t, b=b: wait_store(t - 2, b))
                transpose(b)
                fire_store(t, b)
            return _

        lax.fori_loop(0, BPW // 2, body, None)
        wait_store(BPW - 2, 0)
        wait_store(BPW - 1, 1)

    return k(trip_t, r2)


def kernel(triples, emb_table):
    r2 = emb_table[:NUM_ENT].reshape(NUM_PAIRS_TBL, 128)
    trip_t = jnp.transpose(triples, (2, 1, 0))
    o2 = _gather_native(trip_t, r2)
    return jnp.transpose(o2, (3, 0, 1, 2))


# speedup vs baseline: 12.7675x; 1.2084x over previous
import functools

import jax
import jax.numpy as jnp
from jax import lax
from jax.experimental import pallas as pl
from jax.experimental.pallas import tpu as pltpu
from jax.experimental.pallas import tpu_sc as plsc

B = 1024
L = 200
EMBED_DIM = 64
NUM_ENT = 1000000
NUM_PAIRS_TBL = NUM_ENT // 2

NUM_CORES = 2
NUM_SUBCORES = 16
NUM_WORKERS = NUM_CORES * NUM_SUBCORES
LANES = 16

BBLK = 256
NUM_BLOCKS = L * 2 * (B // BBLK)
BPW = NUM_BLOCKS // NUM_WORKERS
NCH = BBLK // LANES


PACK_EBLK = 1024
PACK_GRID = 489
PAIR_H = PACK_EBLK * PACK_GRID


def _pack_pairs_tc(tt):
    def body(x1_ref, x2_ref, o_ref):
        o_ref[:, 0:EMBED_DIM] = x1_ref[...].T
        o_ref[:, EMBED_DIM:128] = x2_ref[...].T

    return pl.pallas_call(
        body,
        grid=(PACK_GRID,),
        in_specs=[
            pl.BlockSpec((EMBED_DIM, PACK_EBLK), lambda i: (0, i)),
            pl.BlockSpec(
                (EMBED_DIM, PACK_EBLK),
                lambda i: (0, jnp.minimum(i + PACK_GRID, 976)),
            ),
        ],
        out_specs=pl.BlockSpec((PACK_EBLK, 128), lambda i: (i, 0)),
        out_shape=jax.ShapeDtypeStruct((PAIR_H, 128), jnp.float32),
    )(tt, tt)


def _gather_native(trip_t, r2):
    mesh = plsc.VectorSubcoreMesh(core_axis_name="c", subcore_axis_name="s")

    @functools.partial(
        pl.kernel,
        mesh=mesh,
        out_type=jax.ShapeDtypeStruct((L, 2, EMBED_DIM, B), jnp.float32),
        compiler_params=pltpu.CompilerParams(
            needs_layout_passes=False, use_tc_tiling_on_sc=True
        ),
        scratch_types=[
            pltpu.VMEM((2, 2, 128), jnp.int32),
            pltpu.VMEM((2, 2, 128), jnp.int32),
            pltpu.VMEM((2, BBLK), jnp.int32),
            pltpu.VMEM((2, BBLK, 128), jnp.float32),
            pltpu.VMEM((2, 2 * EMBED_DIM, 128), jnp.float32),
            pltpu.SemaphoreType.DMA,
            pltpu.SemaphoreType.DMA,
            pltpu.SemaphoreType.DMA,
            pltpu.SemaphoreType.DMA,
            pltpu.SemaphoreType.DMA,
        ],
    )
    def k(trip_hbm, r2_hbm, o2_hbm, idxr_v, idxp_v, cb_v, g_v, o_v,
          isem0, isem1, gsem, osem0, osem1):
        wid = lax.axis_index("s") * NUM_CORES + lax.axis_index("c")
        lane = lax.iota(jnp.int32, LANES)
        isems = (isem0, isem1)
        osems = (osem0, osem1)

        def parts(t):
            bid = wid * BPW + t
            return bid // 8, (bid // 4) & 1, bid & 3

        def fire_idx(t, b):
            l, j, bb = parts(t)
            for h in range(2):
                pltpu.async_copy(
                    trip_hbm.at[2 * j, l, pl.ds(BBLK * bb + 128 * h, 128)],
                    idxr_v.at[b, h], isems[b],
                )

        def wait_idx(b):
            for h in range(2):
                pltpu.make_async_copy(
                    trip_hbm.at[0, 0, pl.ds(0, 128)], idxr_v.at[b, h], isems[b]
                ).wait()

        def prep_fire_gather(b):
            for h in range(2):
                for c in range(8):
                    v = idxr_v[b, h, pl.ds(c * LANES, LANES)]
                    ge = (v >= PAIR_H).astype(jnp.int32)
                    idxp_v[b, h, pl.ds(c * LANES, LANES)] = v - ge * PAIR_H
                    cb_v[b, pl.ds(h * 128 + c * LANES, LANES)] = ge << 6
            for h in range(2):
                pltpu.async_copy(
                    r2_hbm.at[idxp_v.at[b, h]],
                    g_v.at[b, pl.ds(h * 128, 128), :], gsem,
                )

        def wait_gather(b):
            for h in range(2):
                pltpu.make_async_copy(
                    r2_hbm.at[pl.ds(0, 128), :],
                    g_v.at[b, pl.ds(h * 128, 128), :], gsem,
                ).wait()

        zero16 = lane * 0

        def transpose(b):
            def chunk(c, _):
                e16 = lane + c * LANES
                cb16 = cb_v[b, pl.ds(c * LANES, LANES)]
                el16 = e16 & 127
                eh64 = (c // 8) * 64
                gbase = (e16 << 7) + cb16 + b * (BBLK * 128)
                obase = el16 + (b * 128 + eh64) * 128
                for dc in range(EMBED_DIM // LANES):
                    d0 = dc * LANES
                    rbase = gbase + d0
                    wbase = obase + (d0 << 7)
                    rot = lane
                    for j in range(LANES):
                        val = plsc.load_gather(
                            g_v, [zero16, zero16, rbase + rot]
                        )
                        plsc.store_scatter(
                            o_v, [zero16, zero16, wbase + (rot << 7)], val
                        )
                        rot = (rot + 1) & 15
                return _
            lax.fori_loop(0, NCH, chunk, None)

        def fire_store(t, b):
            l, j, bb = parts(t)
            for eh in range(2):
                pltpu.async_copy(
                    o_v.at[b, pl.ds(eh * EMBED_DIM, EMBED_DIM)],
                    o2_hbm.at[l, j, :, pl.ds(BBLK * bb + 128 * eh, 128)],
                    osems[b],
                )

        def wait_store(t, b):
            l, j, bb = parts(t)
            for eh in range(2):
                pltpu.make_async_copy(
                    o_v.at[b, pl.ds(eh * EMBED_DIM, EMBED_DIM)],
                    o2_hbm.at[l, j, :, pl.ds(BBLK * bb + 128 * eh, 128)],
                    osems[b],
                ).wait()

        fire_idx(0, 0)
        fire_idx(1, 1)
        wait_idx(0)
        prep_fire_gather(0)

        def body(u, _):
            for b in range(2):
                t = 2 * u + b
                nb = 1 - b
                wait_gather(b)
                pl.when(t + 2 < BPW)(lambda t=t, b=b: fire_idx(t + 2, b))

                def mid(t=t, nb=nb):
                    wait_idx(nb)
                    prep_fire_gather(nb)
                pl.when(t + 1 < BPW)(mid)
                pl.when(t >= 2)(lambda t=t, b=b: wait_store(t - 2, b))
                transpose(b)
                fire_store(t, b)
            return _

        lax.fori_loop(0, BPW // 2, body, None)
        wait_store(BPW - 2, 0)
        wait_store(BPW - 1, 1)

    return k(trip_t, r2)


def kernel(triples, emb_table):
    r2 = _pack_pairs_tc(emb_table.T)
    trip_t = jnp.transpose(triples, (2, 1, 0))
    o2 = _gather_native(trip_t, r2)
    return jnp.transpose(o2, (3, 0, 1, 2))


# speedup vs baseline: 17.1132x; 1.3404x over previous
import functools

import jax
import jax.numpy as jnp
from jax import lax
from jax.experimental import pallas as pl
from jax.experimental.pallas import tpu as pltpu
from jax.experimental.pallas import tpu_sc as plsc

B = 1024
L = 200
EMBED_DIM = 64
NUM_ENT = 1000000
NUM_PAIRS_TBL = NUM_ENT // 2

NUM_CORES = 2
NUM_SUBCORES = 16
NUM_WORKERS = NUM_CORES * NUM_SUBCORES
LANES = 16

BBLK = 256
NUM_BLOCKS = L * 2 * (B // BBLK)
BPW = NUM_BLOCKS // NUM_WORKERS
NCH = BBLK // LANES


PACK_EBLK = 4096
PACK_GRID = 123
PAIR_H = PACK_EBLK * PACK_GRID
PACK_LAST_BLK = 244


def _pack_pairs_tc(tt):
    def body(x1_ref, x2_ref, o_ref):
        o_ref[:, 0:EMBED_DIM] = x1_ref[...].T
        o_ref[:, EMBED_DIM:128] = x2_ref[...].T

    return pl.pallas_call(
        body,
        grid=(PACK_GRID,),
        in_specs=[
            pl.BlockSpec((EMBED_DIM, PACK_EBLK), lambda i: (0, i)),
            pl.BlockSpec(
                (EMBED_DIM, PACK_EBLK),
                lambda i: (0, jnp.minimum(i + PACK_GRID, PACK_LAST_BLK)),
            ),
        ],
        out_specs=pl.BlockSpec((PACK_EBLK, 128), lambda i: (i, 0)),
        out_shape=jax.ShapeDtypeStruct((PAIR_H, 128), jnp.float32),
    )(tt, tt)


def _gather_native(trip_t, r2):
    mesh = plsc.VectorSubcoreMesh(core_axis_name="c", subcore_axis_name="s")

    @functools.partial(
        pl.kernel,
        mesh=mesh,
        out_type=jax.ShapeDtypeStruct((L, 2, EMBED_DIM, B), jnp.float32),
        compiler_params=pltpu.CompilerParams(
            needs_layout_passes=False, use_tc_tiling_on_sc=True
        ),
        scratch_types=[
            pltpu.VMEM((2, 2, 128), jnp.int32),
            pltpu.VMEM((2, 2, 128), jnp.int32),
            pltpu.VMEM((2, BBLK), jnp.int32),
            pltpu.VMEM((2, BBLK, 128), jnp.float32),
            pltpu.VMEM((2, 2 * EMBED_DIM, 128), jnp.float32),
            pltpu.SemaphoreType.DMA,
            pltpu.SemaphoreType.DMA,
            pltpu.SemaphoreType.DMA,
            pltpu.SemaphoreType.DMA,
            pltpu.SemaphoreType.DMA,
        ],
    )
    def k(trip_hbm, r2_hbm, o2_hbm, idxr_v, idxp_v, cb_v, g_v, o_v,
          isem0, isem1, gsem, osem0, osem1):
        wid = lax.axis_index("s") * NUM_CORES + lax.axis_index("c")
        lane = lax.iota(jnp.int32, LANES)
        isems = (isem0, isem1)
        osems = (osem0, osem1)

        def parts(t):
            bid = wid * BPW + t
            return bid // 8, (bid // 4) & 1, bid & 3

        def fire_idx(t, b):
            l, j, bb = parts(t)
            for h in range(2):
                pltpu.async_copy(
                    trip_hbm.at[2 * j, l, pl.ds(BBLK * bb + 128 * h, 128)],
                    idxr_v.at[b, h], isems[b],
                )

        def wait_idx(b):
            for h in range(2):
                pltpu.make_async_copy(
                    trip_hbm.at[0, 0, pl.ds(0, 128)], idxr_v.at[b, h], isems[b]
                ).wait()

        def prep_fire_gather(b):
            for h in range(2):
                for c in range(8):
                    v = idxr_v[b, h, pl.ds(c * LANES, LANES)]
                    ge = (v >= PAIR_H).astype(jnp.int32)
                    idxp_v[b, h, pl.ds(c * LANES, LANES)] = v - ge * PAIR_H
                    cb_v[b, pl.ds(h * 128 + c * LANES, LANES)] = ge << 6
            for h in range(2):
                pltpu.async_copy(
                    r2_hbm.at[idxp_v.at[b, h]],
                    g_v.at[b, pl.ds(h * 128, 128), :], gsem,
                )

        def wait_gather(b):
            for h in range(2):
                pltpu.make_async_copy(
                    r2_hbm.at[pl.ds(0, 128), :],
                    g_v.at[b, pl.ds(h * 128, 128), :], gsem,
                ).wait()

        zero16 = lane * 0
        rot_list = [(lane + j) & 15 for j in range(LANES)]
        rots_list = [r << 7 for r in rot_list]

        def transpose(b):
            def chunk(c, _):
                e16 = lane + c * LANES
                cb16 = cb_v[b, pl.ds(c * LANES, LANES)]
                el16 = e16 & 127
                eh64 = (c // 8) * 64
                gbase = (e16 << 7) + cb16 + b * (BBLK * 128)
                obase = el16 + (b * 128 + eh64) * 128
                for dc in range(EMBED_DIM // LANES):
                    d0 = dc * LANES
                    rbase = gbase + d0
                    wbase = obase + (d0 << 7)
                    for j in range(LANES):
                        val = plsc.load_gather(
                            g_v, [zero16, zero16, rbase + rot_list[j]]
                        )
                        plsc.store_scatter(
                            o_v, [zero16, zero16, wbase + rots_list[j]], val
                        )
                return _
            lax.fori_loop(0, NCH, chunk, None)

        def fire_store(t, b):
            l, j, bb = parts(t)
            for eh in range(2):
                pltpu.async_copy(
                    o_v.at[b, pl.ds(eh * EMBED_DIM, EMBED_DIM)],
                    o2_hbm.at[l, j, :, pl.ds(BBLK * bb + 128 * eh, 128)],
                    osems[b],
                )

        def wait_store(t, b):
            l, j, bb = parts(t)
            for eh in range(2):
                pltpu.make_async_copy(
                    o_v.at[b, pl.ds(eh * EMBED_DIM, EMBED_DIM)],
                    o2_hbm.at[l, j, :, pl.ds(BBLK * bb + 128 * eh, 128)],
                    osems[b],
                ).wait()

        fire_idx(0, 0)
        fire_idx(1, 1)
        wait_idx(0)
        prep_fire_gather(0)

        def body(u, _):
            for b in range(2):
                t = 2 * u + b
                nb = 1 - b
                wait_gather(b)
                pl.when(t + 2 < BPW)(lambda t=t, b=b: fire_idx(t + 2, b))

                def mid(t=t, nb=nb):
                    wait_idx(nb)
                    prep_fire_gather(nb)
                pl.when(t + 1 < BPW)(mid)
                pl.when(t >= 2)(lambda t=t, b=b: wait_store(t - 2, b))
                transpose(b)
                fire_store(t, b)
            return _

        lax.fori_loop(0, BPW // 2, body, None)
        wait_store(BPW - 2, 0)
        wait_store(BPW - 1, 1)

    return k(trip_t, r2)


def kernel(triples, emb_table):
    r2 = _pack_pairs_tc(emb_table.T)
    trip_t = jnp.transpose(triples, (2, 1, 0))
    o2 = _gather_native(trip_t, r2)
    return jnp.transpose(o2, (3, 0, 1, 2))


# speedup vs baseline: 18.1417x; 1.0601x over previous
import functools

import jax
import jax.numpy as jnp
from jax import lax
from jax.experimental import pallas as pl
from jax.experimental.pallas import tpu as pltpu
from jax.experimental.pallas import tpu_sc as plsc

B = 1024
L = 200
EMBED_DIM = 64
NUM_ENT = 1000000
NUM_PAIRS_TBL = NUM_ENT // 2

NUM_CORES = 2
NUM_SUBCORES = 16
NUM_WORKERS = NUM_CORES * NUM_SUBCORES
LANES = 16

BBLK = 256
NUM_BLOCKS = L * 2 * (B // BBLK)
BPW = NUM_BLOCKS // NUM_WORKERS
NCH = BBLK // LANES


PACK_EBLK = 8192
PACK_GRID = 62
PAIR_H = PACK_EBLK * PACK_GRID
PACK_LAST_BLK = 122


def _pack_pairs_tc(tt):
    def body(x1_ref, x2_ref, o_ref):
        o_ref[:, 0:EMBED_DIM] = x1_ref[...].T
        o_ref[:, EMBED_DIM:128] = x2_ref[...].T

    return pl.pallas_call(
        body,
        grid=(PACK_GRID,),
        in_specs=[
            pl.BlockSpec((EMBED_DIM, PACK_EBLK), lambda i: (0, i)),
            pl.BlockSpec(
                (EMBED_DIM, PACK_EBLK),
                lambda i: (0, jnp.minimum(i + PACK_GRID, PACK_LAST_BLK)),
            ),
        ],
        out_specs=pl.BlockSpec((PACK_EBLK, 128), lambda i: (i, 0)),
        out_shape=jax.ShapeDtypeStruct((PAIR_H, 128), jnp.float32),
    )(tt, tt)


def _gather_native(trip_t, r2):
    mesh = plsc.VectorSubcoreMesh(core_axis_name="c", subcore_axis_name="s")

    @functools.partial(
        pl.kernel,
        mesh=mesh,
        out_type=jax.ShapeDtypeStruct((L, 2, EMBED_DIM, B), jnp.float32),
        compiler_params=pltpu.CompilerParams(
            needs_layout_passes=False, use_tc_tiling_on_sc=True
        ),
        scratch_types=[
            pltpu.VMEM((2, 2, 128), jnp.int32),
            pltpu.VMEM((2, 2, 128), jnp.int32),
            pltpu.VMEM((2, BBLK), jnp.int32),
            pltpu.VMEM((2, BBLK, 128), jnp.float32),
            pltpu.VMEM((2, 2 * EMBED_DIM, 128), jnp.float32),
            pltpu.SemaphoreType.DMA,
            pltpu.SemaphoreType.DMA,
            pltpu.SemaphoreType.DMA,
            pltpu.SemaphoreType.DMA,
            pltpu.SemaphoreType.DMA,
        ],
    )
    def k(trip_hbm, r2_hbm, o2_hbm, idxr_v, idxp_v, cb_v, g_v, o_v,
          isem0, isem1, gsem, osem0, osem1):
        wid = lax.axis_index("s") * NUM_CORES + lax.axis_index("c")
        lane = lax.iota(jnp.int32, LANES)
        isems = (isem0, isem1)
        osems = (osem0, osem1)

        def parts(t):
            bid = wid * BPW + t
            return bid // 8, (bid // 4) & 1, bid & 3

        def fire_idx(t, b):
            l, j, bb = parts(t)
            for h in range(2):
                pltpu.async_copy(
                    trip_hbm.at[2 * j, l, pl.ds(BBLK * bb + 128 * h, 128)],
                    idxr_v.at[b, h], isems[b],
                )

        def wait_idx(b):
            for h in range(2):
                pltpu.make_async_copy(
                    trip_hbm.at[0, 0, pl.ds(0, 128)], idxr_v.at[b, h], isems[b]
                ).wait()

        def prep_fire_gather(b):
            for h in range(2):
                for c in range(8):
                    v = idxr_v[b, h, pl.ds(c * LANES, LANES)]
                    ge = (v >= PAIR_H).astype(jnp.int32)
                    idxp_v[b, h, pl.ds(c * LANES, LANES)] = v - ge * PAIR_H
                    cb_v[b, pl.ds(h * 128 + c * LANES, LANES)] = ge << 6
            for h in range(2):
                pltpu.async_copy(
                    r2_hbm.at[idxp_v.at[b, h]],
                    g_v.at[b, pl.ds(h * 128, 128), :], gsem,
                )

        def wait_gather(b):
            for h in range(2):
                pltpu.make_async_copy(
                    r2_hbm.at[pl.ds(0, 128), :],
                    g_v.at[b, pl.ds(h * 128, 128), :], gsem,
                ).wait()

        zero16 = lane * 0
        rot_list = [(lane + j) & 15 for j in range(LANES)]
        rots_list = [r << 7 for r in rot_list]

        def transpose(b):
            def chunk(c, _):
                e16 = lane + c * LANES
                cb16 = cb_v[b, pl.ds(c * LANES, LANES)]
                el16 = e16 & 127
                eh64 = (c // 8) * 64
                gbase = (e16 << 7) + cb16 + b * (BBLK * 128)
                obase = el16 + (b * 128 + eh64) * 128
                for dc in range(EMBED_DIM // LANES):
                    d0 = dc * LANES
                    rbase = gbase + d0
                    wbase = obase + (d0 << 7)
                    for j in range(LANES):
                        val = plsc.load_gather(
                            g_v, [zero16, zero16, rbase + rot_list[j]]
                        )
                        plsc.store_scatter(
                            o_v, [zero16, zero16, wbase + rots_list[j]], val
                        )
                return _
            lax.fori_loop(0, NCH, chunk, None)

        def fire_store(t, b):
            l, j, bb = parts(t)
            for eh in range(2):
                pltpu.async_copy(
                    o_v.at[b, pl.ds(eh * EMBED_DIM, EMBED_DIM)],
                    o2_hbm.at[l, j, :, pl.ds(BBLK * bb + 128 * eh, 128)],
                    osems[b],
                )

        def wait_store(t, b):
            l, j, bb = parts(t)
            for eh in range(2):
                pltpu.make_async_copy(
                    o_v.at[b, pl.ds(eh * EMBED_DIM, EMBED_DIM)],
                    o2_hbm.at[l, j, :, pl.ds(BBLK * bb + 128 * eh, 128)],
                    osems[b],
                ).wait()

        fire_idx(0, 0)
        fire_idx(1, 1)
        wait_idx(0)
        prep_fire_gather(0)

        def body(u, _):
            for b in range(2):
                t = 2 * u + b
                nb = 1 - b
                wait_gather(b)
                pl.when(t + 2 < BPW)(lambda t=t, b=b: fire_idx(t + 2, b))

                def mid(t=t, nb=nb):
                    wait_idx(nb)
                    prep_fire_gather(nb)
                pl.when(t + 1 < BPW)(mid)
                pl.when(t >= 2)(lambda t=t, b=b: wait_store(t - 2, b))
                transpose(b)
                fire_store(t, b)
            return _

        lax.fori_loop(0, BPW // 2, body, None)
        wait_store(BPW - 2, 0)
        wait_store(BPW - 1, 1)

    return k(trip_t, r2)


def kernel(triples, emb_table):
    r2 = _pack_pairs_tc(emb_table.T)
    trip_t = jnp.transpose(triples, (2, 1, 0))
    o2 = _gather_native(trip_t, r2)
    return jnp.transpose(o2, (3, 0, 1, 2))
